# baseline (speedup 1.0000x reference)
import jax
import jax.numpy as jnp
from jax.experimental import pallas as pl
from jax.experimental.pallas import tpu as pltpu

_B, _S, _H = 2, 2048, 1024
_MS, _CR = 512, 4
_HH = _H // 2
_BLK = 512
_CBLK = _BLK // _CR


def _silu(x):
    return x * jax.nn.sigmoid(x)


def _dot(a, b):
    return jnp.dot(a, b, preferred_element_type=jnp.float32)


def _fused_body(hs_ref,
                rg_w1_ref, rg_b1_ref, rg_w2_ref, rg_b2_ref,
                wg_w1_ref, wg_b1_ref, wg_w2_ref, wg_b2_ref,
                comp_w1_ref, comp_b1_ref, comp_w2_ref, comp_b2_ref,
                o_b_ref, ln_g_ref, ln_b_ref,
                out_ref, bank_ref,
                gw_bf, cw1_bf, cw2_bf):
    @pl.when(pl.program_id(0) == 0)
    def _init_bf16_weights():
        gw_bf[:, :_HH] = rg_w1_ref[:_H, :].astype(jnp.bfloat16)
        gw_bf[:, _HH:] = wg_w1_ref[...].astype(jnp.bfloat16)
        cw1_bf[...] = comp_w1_ref[...].astype(jnp.bfloat16)
        cw2_bf[...] = comp_w2_ref[...].astype(jnp.bfloat16)

    x = hs_ref[...]
    xb = x.astype(jnp.bfloat16)
    ob = o_b_ref[...][None, :]

    g = _dot(xb, gw_bf[...])

    c1 = _dot(ob, rg_w1_ref[_H:, :]) + rg_b1_ref[...][None, :]
    a1 = _silu(g[:, :_HH] + c1)
    rgate = jax.nn.sigmoid(_dot(a1, rg_w2_ref[...]) + rg_b2_ref[...][None, :])
    out = x + rgate * ob
    mu = jnp.mean(out, axis=-1, keepdims=True)
    d = out - mu
    var = jnp.mean(d * d, axis=-1, keepdims=True)
    out_ref[...] = (d * jax.lax.rsqrt(var + 1e-5) * ln_g_ref[...][None, :]
                    + ln_b_ref[...][None, :])

    a2 = _silu(g[:, _HH:] + wg_b1_ref[...][None, :])
    wgate = jax.nn.sigmoid(_dot(a2, wg_w2_ref[...]) + wg_b2_ref[...][None, :])
    row = jax.lax.broadcasted_iota(jnp.int32, (_CBLK, _BLK), 0)
    col = jax.lax.broadcasted_iota(jnp.int32, (_CBLK, _BLK), 1)
    pool = jnp.where(col // _CR == row, 1.0 / _CR, 0.0)
    gmean = _dot(pool, wgate)

    x4 = xb.reshape(_CBLK, _CR, _H)
    acc = comp_b1_ref[...][None, :] + jnp.zeros((_CBLK, _H), jnp.float32)
    for j in range(_CR):
        acc = acc + _dot(x4[:, j, :], cw1_bf[pl.ds(j * _H, _H), :])
    h1 = _silu(acc)
    comp = _dot(h1.astype(jnp.bfloat16), cw2_bf[...]) + comp_b2_ref[...][None, :]
    bank_ref[...] = comp * gmean


def kernel(hidden_states, comp_w1, comp_b1, comp_w2, comp_b2,
           q_w, q_b, k_w, k_b, v_w, v_b, o_w, o_b,
           wg_w1, wg_b1, wg_w2, wg_b2, rg_w1, rg_b1, rg_w2, rg_b2,
           ln_g, ln_b):
    b, s, h = hidden_states.shape
    n_tok = b * s
    hs2 = hidden_states.reshape(n_tok, h)

    grid = (n_tok // _BLK,)
    full = lambda arr: pl.BlockSpec(arr.shape, lambda i: (0,) * arr.ndim)

    w_args = (rg_w1, rg_b1, rg_w2, rg_b2,
              wg_w1, wg_b1, wg_w2, wg_b2,
              comp_w1, comp_b1, comp_w2, comp_b2,
              o_b, ln_g, ln_b)

    out2, bank2 = pl.pallas_call(
        _fused_body,
        grid=grid,
        in_specs=[pl.BlockSpec((_BLK, h), lambda i: (i, 0)),
                  *(full(a) for a in w_args)],
        out_specs=[pl.BlockSpec((_BLK, h), lambda i: (i, 0)),
                   pl.BlockSpec((_CBLK, h), lambda i: (i, 0))],
        out_shape=[jax.ShapeDtypeStruct((n_tok, h), jnp.float32),
                   jax.ShapeDtypeStruct((n_tok // _CR, h), jnp.float32)],
        scratch_shapes=[pltpu.VMEM((h, h), jnp.bfloat16),
                        pltpu.VMEM((_CR * h, h), jnp.bfloat16),
                        pltpu.VMEM((h, h), jnp.bfloat16)],
    )(hs2, *w_args)

    output = out2.reshape(b, s, h)
    memory_bank = bank2.reshape(b, s // _CR, h)
    memory_mask = jnp.ones((b, _MS), dtype=bool)
    return (output, memory_bank, memory_mask)


# speedup vs baseline: 1.0639x; 1.0639x over previous
import jax
import jax.numpy as jnp
from jax.experimental import pallas as pl
from jax.experimental.pallas import tpu as pltpu

_B, _S, _H = 2, 2048, 1024
_MS, _CR = 512, 4
_BLK = 512
_CBLK = _BLK // _CR


def _silu(x):
    return x * jax.nn.sigmoid(x)


def _dot(a, b):
    return jnp.dot(a, b, preferred_element_type=jnp.float32,
                   precision=jax.lax.Precision.DEFAULT)


def _fused_body(hs_ref,
                rg_w1_ref, rg_b1_ref, rg_w2_ref, rg_b2_ref,
                wg_w1_ref, wg_b1_ref, wg_w2_ref, wg_b2_ref,
                comp_w1_ref, comp_b1_ref, comp_w2_ref, comp_b2_ref,
                o_b_ref, ln_g_ref, ln_b_ref,
                out_ref, bank_ref, cw1_bf):
    @pl.when(pl.program_id(0) == 0)
    def _init_bf16_weights():
        cw1_bf[...] = comp_w1_ref[...].astype(jnp.bfloat16)

    x = hs_ref[...]
    ob = o_b_ref[...][None, :]

    c1 = _dot(ob, rg_w1_ref[_H:, :]) + rg_b1_ref[...][None, :]
    a1 = _silu(_dot(x, rg_w1_ref[:_H, :]) + c1)
    rgate = jax.nn.sigmoid(_dot(a1, rg_w2_ref[...]) + rg_b2_ref[...][None, :])
    out = x + rgate * ob
    mu = jnp.mean(out, axis=-1, keepdims=True)
    d = out - mu
    var = jnp.mean(d * d, axis=-1, keepdims=True)
    out_ref[...] = (d * jax.lax.rsqrt(var + 1e-5) * ln_g_ref[...][None, :]
                    + ln_b_ref[...][None, :])

    a2 = _silu(_dot(x, wg_w1_ref[...]) + wg_b1_ref[...][None, :])
    wgate = jax.nn.sigmoid(_dot(a2, wg_w2_ref[...]) + wg_b2_ref[...][None, :])
    row = jax.lax.broadcasted_iota(jnp.int32, (_CBLK, _BLK), 0)
    col = jax.lax.broadcasted_iota(jnp.int32, (_CBLK, _BLK), 1)
    pool = jnp.where(col // _CR == row, 1.0 / _CR, 0.0)
    gmean = _dot(pool, wgate)

    x4 = x.astype(jnp.bfloat16).reshape(_CBLK, _CR, _H)
    acc = comp_b1_ref[...][None, :] + jnp.zeros((_CBLK, _H), jnp.float32)
    for j in range(_CR):
        acc = acc + _dot(x4[:, j, :], cw1_bf[pl.ds(j * _H, _H), :])
    h1 = _silu(acc)
    comp = _dot(h1, comp_w2_ref[...]) + comp_b2_ref[...][None, :]
    bank_ref[...] = comp * gmean


def kernel(hidden_states, comp_w1, comp_b1, comp_w2, comp_b2,
           q_w, q_b, k_w, k_b, v_w, v_b, o_w, o_b,
           wg_w1, wg_b1, wg_w2, wg_b2, rg_w1, rg_b1, rg_w2, rg_b2,
           ln_g, ln_b):
    b, s, h = hidden_states.shape
    n_tok = b * s
    hs2 = hidden_states.reshape(n_tok, h)

    grid = (n_tok // _BLK,)
    full = lambda arr: pl.BlockSpec(arr.shape, lambda i: (0,) * arr.ndim)

    w_args = (rg_w1, rg_b1, rg_w2, rg_b2,
              wg_w1, wg_b1, wg_w2, wg_b2,
              comp_w1, comp_b1, comp_w2, comp_b2,
              o_b, ln_g, ln_b)

    out2, bank2 = pl.pallas_call(
        _fused_body,
        grid=grid,
        in_specs=[pl.BlockSpec((_BLK, h), lambda i: (i, 0)),
                  *(full(a) for a in w_args)],
        out_specs=[pl.BlockSpec((_BLK, h), lambda i: (i, 0)),
                   pl.BlockSpec((_CBLK, h), lambda i: (i, 0))],
        out_shape=[jax.ShapeDtypeStruct((n_tok, h), jnp.float32),
                   jax.ShapeDtypeStruct((n_tok // _CR, h), jnp.float32)],
        scratch_shapes=[pltpu.VMEM((_CR * h, h), jnp.bfloat16)],
    )(hs2, *w_args)

    output = out2.reshape(b, s, h)
    memory_bank = bank2.reshape(b, s // _CR, h)
    memory_mask = jnp.ones((b, _MS), dtype=bool)
    return (output, memory_bank, memory_mask)


# speedup vs baseline: 1.1640x; 1.0941x over previous
import jax
import jax.numpy as jnp
from jax.experimental import pallas as pl

_B, _S, _H = 2, 2048, 1024
_MS, _CR = 512, 4
_BLK = 1024
_CBLK = _BLK // _CR


def _silu(x):
    return x * jax.nn.sigmoid(x)


def _dot(a, b):
    return jnp.dot(a, b, preferred_element_type=jnp.float32)


def _fused_body(hs_ref,
                rg_w1a_ref, rg_w1b_ref, rg_b1_ref, rg_w2_ref, rg_b2_ref,
                wg_w1_ref, wg_b1_ref, wg_w2_ref, wg_b2_ref,
                cw1_0, cw1_1, cw1_2, cw1_3,
                comp_b1_ref, comp_w2_ref, comp_b2_ref,
                o_b_ref, ln_g_ref, ln_b_ref,
                out_ref, bank_ref):
    x = hs_ref[...]
    ob = o_b_ref[...][None, :]

    c1 = _dot(ob, rg_w1b_ref[...]) + rg_b1_ref[...][None, :]
    a1 = _silu(_dot(x, rg_w1a_ref[...]) + c1)
    rgate = jax.nn.sigmoid(_dot(a1, rg_w2_ref[...]) + rg_b2_ref[...][None, :])
    out = x + rgate * ob
    mu = jnp.mean(out, axis=-1, keepdims=True)
    d = out - mu
    var = jnp.mean(d * d, axis=-1, keepdims=True)
    out_ref[...] = (d * jax.lax.rsqrt(var + 1e-5) * ln_g_ref[...][None, :]
                    + ln_b_ref[...][None, :])

    a2 = _silu(_dot(x, wg_w1_ref[...]) + wg_b1_ref[...][None, :])
    wgate = jax.nn.sigmoid(_dot(a2, wg_w2_ref[...]) + wg_b2_ref[...][None, :])
    row = jax.lax.broadcasted_iota(jnp.int32, (_CBLK, _BLK), 0)
    col = jax.lax.broadcasted_iota(jnp.int32, (_CBLK, _BLK), 1)
    pool = jnp.where(col // _CR == row, 1.0 / _CR, 0.0)
    gmean = _dot(pool, wgate)

    x4 = x.reshape(_CBLK, _CR, _H)
    acc = comp_b1_ref[...][None, :] + jnp.zeros((_CBLK, _H), jnp.float32)
    for j, cw in enumerate((cw1_0, cw1_1, cw1_2, cw1_3)):
        acc = acc + _dot(x4[:, j, :], cw[...])
    h1 = _silu(acc)
    comp = _dot(h1, comp_w2_ref[...]) + comp_b2_ref[...][None, :]
    bank_ref[...] = comp * gmean


def kernel(hidden_states, comp_w1, comp_b1, comp_w2, comp_b2,
           q_w, q_b, k_w, k_b, v_w, v_b, o_w, o_b,
           wg_w1, wg_b1, wg_w2, wg_b2, rg_w1, rg_b1, rg_w2, rg_b2,
           ln_g, ln_b):
    b, s, h = hidden_states.shape
    n_tok = b * s
    hs2 = hidden_states.reshape(n_tok, h)

    grid = (n_tok // _BLK,)
    full = lambda arr: pl.BlockSpec(arr.shape, lambda i: (0,) * arr.ndim)

    def chunk_spec(shape, blk_idx):
        return pl.BlockSpec(shape, lambda i, j=blk_idx: (j, 0))

    out2, bank2 = pl.pallas_call(
        _fused_body,
        grid=grid,
        in_specs=[pl.BlockSpec((_BLK, h), lambda i: (i, 0)),
                  chunk_spec((h, h // 2), 0),
                  chunk_spec((h, h // 2), 1),
                  full(rg_b1), full(rg_w2), full(rg_b2),
                  full(wg_w1), full(wg_b1), full(wg_w2), full(wg_b2),
                  chunk_spec((h, h), 0),
                  chunk_spec((h, h), 1),
                  chunk_spec((h, h), 2),
                  chunk_spec((h, h), 3),
                  full(comp_b1), full(comp_w2), full(comp_b2),
                  full(o_b), full(ln_g), full(ln_b)],
        out_specs=[pl.BlockSpec((_BLK, h), lambda i: (i, 0)),
                   pl.BlockSpec((_CBLK, h), lambda i: (i, 0))],
        out_shape=[jax.ShapeDtypeStruct((n_tok, h), jnp.float32),
                   jax.ShapeDtypeStruct((n_tok // _CR, h), jnp.float32)],
    )(hs2, rg_w1, rg_w1, rg_b1, rg_w2, rg_b2,
      wg_w1, wg_b1, wg_w2, wg_b2,
      comp_w1, comp_w1, comp_w1, comp_w1,
      comp_b1, comp_w2, comp_b2, o_b, ln_g, ln_b)

    output = out2.reshape(b, s, h)
    memory_bank = bank2.reshape(b, s // _CR, h)
    memory_mask = jnp.ones((b, _MS), dtype=bool)
    return (output, memory_bank, memory_mask)


# speedup vs baseline: 1.1726x; 1.0073x over previous
import jax
import jax.numpy as jnp
from jax.experimental import pallas as pl

_B, _S, _H = 2, 2048, 1024
_MS, _CR = 512, 4
_BLK = 1024
_CBLK = _BLK // _CR


def _silu(x):
    return x * jax.nn.sigmoid(x)


def _dot(a, b):
    return jnp.dot(a, b, preferred_element_type=jnp.float32)


def _fused_body(hs_ref,
                rg_w1a_ref, rg_w1b_ref, rg_b1_ref, rg_w2_ref, rg_b2_ref,
                wg_w1_ref, wg_b1_ref, wg_w2_ref, wg_b2_ref,
                cw1_0, cw1_1, cw1_2, cw1_3,
                comp_b1_ref, comp_w2_ref, comp_b2_ref,
                o_b_ref, ln_g_ref, ln_b_ref,
                out_ref, bank_ref):
    x = hs_ref[...]
    ob = o_b_ref[...][None, :]

    c1 = _dot(ob, rg_w1b_ref[...]) + rg_b1_ref[...][None, :]
    a1 = _silu(_dot(x, rg_w1a_ref[...]) + c1)
    rgate = jax.nn.sigmoid(_dot(a1, rg_w2_ref[...]) + rg_b2_ref[...][None, :])
    out = x + rgate * ob
    mu = jnp.mean(out, axis=-1, keepdims=True)
    d = out - mu
    var = jnp.mean(d * d, axis=-1, keepdims=True)
    out_ref[...] = (d * jax.lax.rsqrt(var + 1e-5) * ln_g_ref[...][None, :]
                    + ln_b_ref[...][None, :])

    a2 = _silu(_dot(x, wg_w1_ref[...]) + wg_b1_ref[...][None, :])
    wgate = jax.nn.sigmoid(_dot(a2, wg_w2_ref[...]) + wg_b2_ref[...][None, :])
    gmean = jnp.mean(wgate.reshape(_CBLK, _CR), axis=1, keepdims=True)

    x4 = x.reshape(_CBLK, _CR, _H)
    acc = comp_b1_ref[...][None, :] + jnp.zeros((_CBLK, _H), jnp.float32)
    for j, cw in enumerate((cw1_0, cw1_1, cw1_2, cw1_3)):
        acc = acc + _dot(x4[:, j, :], cw[...])
    h1 = _silu(acc)
    comp = _dot(h1, comp_w2_ref[...]) + comp_b2_ref[...][None, :]
    bank_ref[...] = comp * gmean


def kernel(hidden_states, comp_w1, comp_b1, comp_w2, comp_b2,
           q_w, q_b, k_w, k_b, v_w, v_b, o_w, o_b,
           wg_w1, wg_b1, wg_w2, wg_b2, rg_w1, rg_b1, rg_w2, rg_b2,
           ln_g, ln_b):
    b, s, h = hidden_states.shape
    n_tok = b * s
    hs2 = hidden_states.reshape(n_tok, h)

    grid = (n_tok // _BLK,)
    full = lambda arr: pl.BlockSpec(arr.shape, lambda i: (0,) * arr.ndim)

    def chunk_spec(shape, blk_idx):
        return pl.BlockSpec(shape, lambda i, j=blk_idx: (j, 0))

    out2, bank2 = pl.pallas_call(
        _fused_body,
        grid=grid,
        in_specs=[pl.BlockSpec((_BLK, h), lambda i: (i, 0)),
                  chunk_spec((h, h // 2), 0),
                  chunk_spec((h, h // 2), 1),
                  full(rg_b1), full(rg_w2), full(rg_b2),
                  full(wg_w1), full(wg_b1), full(wg_w2), full(wg_b2),
                  chunk_spec((h, h), 0),
                  chunk_spec((h, h), 1),
                  chunk_spec((h, h), 2),
                  chunk_spec((h, h), 3),
                  full(comp_b1), full(comp_w2), full(comp_b2),
                  full(o_b), full(ln_g), full(ln_b)],
        out_specs=[pl.BlockSpec((_BLK, h), lambda i: (i, 0)),
                   pl.BlockSpec((_CBLK, h), lambda i: (i, 0))],
        out_shape=[jax.ShapeDtypeStruct((n_tok, h), jnp.float32),
                   jax.ShapeDtypeStruct((n_tok // _CR, h), jnp.float32)],
    )(hs2, rg_w1, rg_w1, rg_b1, rg_w2, rg_b2,
      wg_w1, wg_b1, wg_w2, wg_b2,
      comp_w1, comp_w1, comp_w1, comp_w1,
      comp_b1, comp_w2, comp_b2, o_b, ln_g, ln_b)

    output = out2.reshape(b, s, h)
    memory_bank = bank2.reshape(b, s // _CR, h)
    memory_mask = jnp.ones((b, _MS), dtype=bool)
    return (output, memory_bank, memory_mask)


# speedup vs baseline: 1.1795x; 1.0059x over previous
import jax
import jax.numpy as jnp
from jax.experimental import pallas as pl
from jax.experimental.pallas import tpu as pltpu

_B, _S, _H = 2, 2048, 1024
_MS, _CR = 512, 4
_BLK = 1024
_CBLK = _BLK // _CR


def _silu(x):
    return x * jax.nn.sigmoid(x)


def _dot(a, b):
    return jnp.dot(a, b, preferred_element_type=jnp.float32)


def _fused_body(hs_ref,
                rg_w1a_ref, rg_w1b_ref, rg_b1_ref, rg_w2_ref, rg_b2_ref,
                wg_w1_ref, wg_b1_ref, wg_w2_ref, wg_b2_ref,
                cw1_0, cw1_1, cw1_2, cw1_3,
                comp_b1_ref, comp_w2_ref, comp_b2_ref,
                o_b_ref, ln_g_ref, ln_b_ref,
                out_ref, bank_ref):
    x = hs_ref[...]
    ob = o_b_ref[...][None, :]

    c1 = _dot(ob, rg_w1b_ref[...]) + rg_b1_ref[...][None, :]
    a1 = _silu(_dot(x, rg_w1a_ref[...]) + c1)
    rgate = jax.nn.sigmoid(_dot(a1, rg_w2_ref[...]) + rg_b2_ref[...][None, :])
    out = x + rgate * ob
    mu = jnp.mean(out, axis=-1, keepdims=True)
    m2 = jnp.mean(out * out, axis=-1, keepdims=True)
    var = m2 - mu * mu
    out_ref[...] = ((out - mu) * jax.lax.rsqrt(var + 1e-5) * ln_g_ref[...][None, :]
                    + ln_b_ref[...][None, :])

    a2 = _silu(_dot(x, wg_w1_ref[...]) + wg_b1_ref[...][None, :])
    wgate = jax.nn.sigmoid(_dot(a2, wg_w2_ref[...]) + wg_b2_ref[...][None, :])
    gmean = jnp.mean(wgate.reshape(_CBLK, _CR), axis=1, keepdims=True)

    x4 = x.reshape(_CBLK, _CR, _H)
    acc = _dot(x4[:, 0, :], cw1_0[...])
    for j, cw in enumerate((cw1_1, cw1_2, cw1_3), start=1):
        acc = acc + _dot(x4[:, j, :], cw[...])
    h1 = _silu(acc + comp_b1_ref[...][None, :])
    comp = _dot(h1, comp_w2_ref[...]) + comp_b2_ref[...][None, :]
    bank_ref[...] = comp * gmean


def kernel(hidden_states, comp_w1, comp_b1, comp_w2, comp_b2,
           q_w, q_b, k_w, k_b, v_w, v_b, o_w, o_b,
           wg_w1, wg_b1, wg_w2, wg_b2, rg_w1, rg_b1, rg_w2, rg_b2,
           ln_g, ln_b):
    b, s, h = hidden_states.shape
    n_tok = b * s
    hs2 = hidden_states.reshape(n_tok, h)

    grid = (n_tok // _BLK,)
    full = lambda arr: pl.BlockSpec(arr.shape, lambda i: (0,) * arr.ndim)

    def chunk_spec(shape, blk_idx):
        return pl.BlockSpec(shape, lambda i, j=blk_idx: (j, 0))

    out2, bank2 = pl.pallas_call(
        _fused_body,
        grid=grid,
        in_specs=[pl.BlockSpec((_BLK, h), lambda i: (i, 0)),
                  chunk_spec((h, h // 2), 0),
                  chunk_spec((h, h // 2), 1),
                  full(rg_b1), full(rg_w2), full(rg_b2),
                  full(wg_w1), full(wg_b1), full(wg_w2), full(wg_b2),
                  chunk_spec((h, h), 0),
                  chunk_spec((h, h), 1),
                  chunk_spec((h, h), 2),
                  chunk_spec((h, h), 3),
                  full(comp_b1), full(comp_w2), full(comp_b2),
                  full(o_b), full(ln_g), full(ln_b)],
        out_specs=[pl.BlockSpec((_BLK, h), lambda i: (i, 0)),
                   pl.BlockSpec((_CBLK, h), lambda i: (i, 0))],
        out_shape=[jax.ShapeDtypeStruct((n_tok, h), jnp.float32),
                   jax.ShapeDtypeStruct((n_tok // _CR, h), jnp.float32)],
    )(hs2, rg_w1, rg_w1, rg_b1, rg_w2, rg_b2,
      wg_w1, wg_b1, wg_w2, wg_b2,
      comp_w1, comp_w1, comp_w1, comp_w1,
      comp_b1, comp_w2, comp_b2, o_b, ln_g, ln_b)

    output = out2.reshape(b, s, h)
    memory_bank = bank2.reshape(b, s // _CR, h)
    memory_mask = jnp.ones((b, _MS), dtype=bool)
    return (output, memory_bank, memory_mask)
